# baseline (speedup 1.0000x reference)
import functools

import jax
import jax.numpy as jnp
from jax import lax
from jax.experimental import pallas as pl
from jax.experimental.pallas import tpu as pltpu
from jax.experimental.pallas import tpu_sc as plsc

N = 10000
D_IN = 128
D_OUT = 256
DQ = 32
L = 16
TILES = 16
NP = 10240
RPT = NP // TILES
EB = 128
NSLOT = 8
XROWS = RPT // 4


def _rsqrt16(d):
    i = lax.bitcast_convert_type(d, jnp.int32)
    y = lax.bitcast_convert_type(jnp.int32(0x5F3759DF) - (i >> 1), jnp.float32)
    for _ in range(3):
        y = y * (1.5 - 0.5 * d * y * y)
    return y


def _sc_aggregate(x_q0, x_q1, x_q2, x_q3, src2, dst2):
    nrow_all = src2.shape[0]
    nbatch = nrow_all // TILES
    assert nbatch % NSLOT == 0
    fl = jnp.float32
    i32 = jnp.int32
    mesh = plsc.VectorSubcoreMesh(core_axis_name="c", subcore_axis_name="s")

    @functools.partial(
        pl.kernel,
        out_type=(
            jax.ShapeDtypeStruct((NP, DQ), fl),
            jax.ShapeDtypeStruct((NP, DQ), fl),
            jax.ShapeDtypeStruct((NP, DQ), fl),
            jax.ShapeDtypeStruct((NP, DQ), fl),
            jax.ShapeDtypeStruct((NP,), fl),
        ),
        mesh=mesh,
        scratch_types=[
            pltpu.VMEM_SHARED((NP, DQ), fl),
            pltpu.VMEM_SHARED((NP, DQ), fl),
            pltpu.VMEM_SHARED((NP,), fl),
            pltpu.VMEM((nbatch, EB), i32),
            pltpu.VMEM((nbatch, EB), i32),
            pltpu.VMEM((XROWS, DQ), fl),
            pltpu.VMEM((RPT,), fl),
            pltpu.VMEM((RPT,), fl),
            [pltpu.VMEM((EB, DQ), fl)] * NSLOT,
            pltpu.VMEM((EB,), fl),
            pltpu.SemaphoreType.DMA,
            pltpu.SemaphoreType.DMA,
            pltpu.SemaphoreType.DMA,
        ],
        compiler_params=pltpu.CompilerParams(use_tc_tiling_on_sc=False),
    )
    def k(x_q0_h, x_q1_h, x_q2_h, x_q3_h, src_h, dst_h,
          agg_q0_h, agg_q1_h, agg_q2_h, agg_q3_h, dis_h,
          agg_sh, xs_sh, deg_sh, src_all, dst_all, xbuf, degl, disl, rows,
          ones, sem_a, sem_g, sem_sc):
        cid = lax.axis_index("c")
        tid = lax.axis_index("s")
        r0 = tid * RPT

        @pl.loop(0, RPT // L)
        def _(i):
            degl[pl.ds(i * L, L)] = jnp.zeros((L,), fl)

        @pl.loop(0, EB // L)
        def _(i):
            ones[pl.ds(i * L, L)] = jnp.ones((L,), fl)

        pltpu.sync_copy(src_h.at[pl.ds(tid * nbatch, nbatch)], src_all)
        pltpu.sync_copy(dst_h.at[pl.ds(tid * nbatch, nbatch)], dst_all)

        pltpu.sync_copy(degl, deg_sh.at[pl.ds(r0, RPT)])
        plsc.subcore_barrier()

        for s in range(NSLOT):
            pltpu.async_copy(ones, deg_sh.at[dst_all.at[s]], sem_a, add=True)

        @pl.loop(0, nbatch)
        def _(i):
            @pl.when(i < nbatch - NSLOT)
            def _():
                pltpu.async_copy(ones, deg_sh.at[dst_all.at[i + NSLOT]],
                                 sem_a, add=True)

            pltpu.make_async_copy(ones, deg_sh.at[dst_all.at[0]], sem_a).wait()

        plsc.subcore_barrier()

        pltpu.sync_copy(deg_sh.at[pl.ds(r0, RPT)], degl)

        @pl.loop(0, RPT // L)
        def _(i):
            disl[pl.ds(i * L, L)] = _rsqrt16(degl[pl.ds(i * L, L)])

        @pl.when(cid == 0)
        def _():
            pltpu.sync_copy(disl, dis_h.at[pl.ds(r0, RPT)])

        def do_pass(x_h, agg_out_h):
            HG = NSLOT // 2

            for h in range(4):
                rh = r0 + h * XROWS

                @pl.loop(0, XROWS)
                def _(r):
                    for j in range(DQ // L):
                        xbuf[r, pl.ds(j * L, L)] = jnp.zeros((L,), fl)

                pltpu.sync_copy(xbuf, agg_sh.at[pl.ds(rh, XROWS)])
                pltpu.sync_copy(x_h.at[pl.ds(rh, XROWS)], xbuf)

                @pl.loop(0, XROWS // L)
                def _(i):
                    dv = disl[pl.ds(h * XROWS + i * L, L)]
                    for kk in range(L):
                        s = dv[kk]
                        r = i * L + kk
                        for j in range(DQ // L):
                            xbuf[r, pl.ds(j * L, L)] = (
                                xbuf[r, pl.ds(j * L, L)] * s)

                pltpu.sync_copy(xbuf, xs_sh.at[pl.ds(rh, XROWS)])

            plsc.subcore_barrier()

            def fire_gather(i, s):
                pltpu.async_copy(xs_sh.at[src_all.at[i]], rows[s], sem_g)

            def wait_gather(s):
                pltpu.make_async_copy(xs_sh.at[src_all.at[0]], rows[s],
                                      sem_g).wait()

            def fire_scatter(i, s):
                pltpu.async_copy(rows[s], agg_sh.at[dst_all.at[i]],
                                 sem_sc, add=True)

            def wait_scatter(s):
                pltpu.make_async_copy(rows[s], agg_sh.at[dst_all.at[0]],
                                      sem_sc).wait()

            for s in range(HG):
                fire_gather(s, s)

            @pl.loop(0, nbatch // NSLOT)
            def _(j):
                for s in range(NSLOT):
                    i = j * NSLOT + s
                    wait_gather(s)
                    fire_scatter(i, s)

                    @pl.when(i >= HG)
                    def _():
                        wait_scatter((s + HG) % NSLOT)

                    @pl.when(i < nbatch - HG)
                    def _():
                        fire_gather(i + HG, (s + HG) % NSLOT)

            for s in range(HG):
                wait_scatter(HG + s)

            plsc.subcore_barrier()

            pltpu.sync_copy(agg_sh.at[pl.ds(r0, RPT)],
                            agg_out_h.at[pl.ds(r0, RPT)])
            plsc.subcore_barrier()

        @pl.when(cid == 0)
        def _():
            do_pass(x_q0_h, agg_q0_h)
            do_pass(x_q1_h, agg_q1_h)

        @pl.when(cid == 1)
        def _():
            do_pass(x_q2_h, agg_q2_h)
            do_pass(x_q3_h, agg_q3_h)

    return k(x_q0, x_q1, x_q2, x_q3, src2, dst2)


def _tc_matmul(a0, a1, a2, a3, dis, W, b):
    BM = 256

    def body(a0_ref, a1_ref, a2_ref, a3_ref, dis_ref, w_ref, b_ref, o_ref):
        xc = jnp.concatenate(
            [a0_ref[...], a1_ref[...], a2_ref[...], a3_ref[...]], axis=1
        ) * dis_ref[...]
        o_ref[...] = (
            jnp.dot(xc, w_ref[...], preferred_element_type=jnp.float32)
            + b_ref[...]
        )

    qspec = pl.BlockSpec((BM, DQ), lambda i: (i, 0))
    return pl.pallas_call(
        body,
        grid=(NP // BM,),
        in_specs=[
            qspec, qspec, qspec, qspec,
            pl.BlockSpec((BM, 1), lambda i: (i, 0)),
            pl.BlockSpec((D_IN, D_OUT), lambda i: (0, 0)),
            pl.BlockSpec((1, D_OUT), lambda i: (0, 0)),
        ],
        out_specs=pl.BlockSpec((BM, D_OUT), lambda i: (i, 0)),
        out_shape=jax.ShapeDtypeStruct((NP, D_OUT), jnp.float32),
    )(a0, a1, a2, a3, dis, W, b.reshape(1, D_OUT))


def kernel(x, edge_index, W, b):
    src = edge_index[0].astype(jnp.int32)
    dst = edge_index[1].astype(jnp.int32)
    loop_idx = jnp.arange(N, dtype=jnp.int32)
    e_tot = N + src.shape[0]
    group = TILES * EB * 24
    e_pad = ((e_tot + group - 1) // group) * group
    padv = jnp.full((e_pad - e_tot,), NP - 1, jnp.int32)
    src2 = jnp.concatenate([src, loop_idx, padv]).reshape(e_pad // EB, EB)
    dst2 = jnp.concatenate([dst, loop_idx, padv]).reshape(e_pad // EB, EB)
    x_pad = jnp.zeros((NP, D_IN), jnp.float32).at[:N].set(x)
    a0, a1, a2, a3, dis = _sc_aggregate(
        x_pad[:, 0 * DQ:1 * DQ], x_pad[:, 1 * DQ:2 * DQ],
        x_pad[:, 2 * DQ:3 * DQ], x_pad[:, 3 * DQ:4 * DQ], src2, dst2)
    out = _tc_matmul(a0, a1, a2, a3, dis.reshape(NP, 1), W, b)
    return out[:N]


# speedup vs baseline: 1.0073x; 1.0073x over previous
import functools

import jax
import jax.numpy as jnp
from jax import lax
from jax.experimental import pallas as pl
from jax.experimental.pallas import tpu as pltpu
from jax.experimental.pallas import tpu_sc as plsc

N = 10000
D_IN = 128
D_OUT = 256
DQ = 32
L = 16
TILES = 16
NP = 10240
RPT = NP // TILES
EB = 128
NSLOT = 4
XROWS = RPT // 4


def _rsqrt16(d):
    i = lax.bitcast_convert_type(d, jnp.int32)
    y = lax.bitcast_convert_type(jnp.int32(0x5F3759DF) - (i >> 1), jnp.float32)
    for _ in range(3):
        y = y * (1.5 - 0.5 * d * y * y)
    return y


def _sc_aggregate(x_q0, x_q1, x_q2, x_q3, src2, dst2):
    nrow_all = src2.shape[0]
    nbatch = nrow_all // TILES
    assert nbatch % NSLOT == 0
    fl = jnp.float32
    i32 = jnp.int32
    mesh = plsc.VectorSubcoreMesh(core_axis_name="c", subcore_axis_name="s")

    @functools.partial(
        pl.kernel,
        out_type=(
            jax.ShapeDtypeStruct((NP, DQ), fl),
            jax.ShapeDtypeStruct((NP, DQ), fl),
            jax.ShapeDtypeStruct((NP, DQ), fl),
            jax.ShapeDtypeStruct((NP, DQ), fl),
            jax.ShapeDtypeStruct((NP,), fl),
        ),
        mesh=mesh,
        scratch_types=[
            pltpu.VMEM_SHARED((NP, DQ), fl),
            pltpu.VMEM_SHARED((NP, DQ), fl),
            pltpu.VMEM_SHARED((NP,), fl),
            pltpu.VMEM((nbatch, EB), i32),
            pltpu.VMEM((nbatch, EB), i32),
            pltpu.VMEM((XROWS, DQ), fl),
            pltpu.VMEM((RPT,), fl),
            pltpu.VMEM((RPT,), fl),
            [pltpu.VMEM((EB, DQ), fl)] * NSLOT,
            pltpu.VMEM((EB,), fl),
            pltpu.SemaphoreType.DMA,
            pltpu.SemaphoreType.DMA,
            pltpu.SemaphoreType.DMA,
        ],
        compiler_params=pltpu.CompilerParams(use_tc_tiling_on_sc=False),
    )
    def k(x_q0_h, x_q1_h, x_q2_h, x_q3_h, src_h, dst_h,
          agg_q0_h, agg_q1_h, agg_q2_h, agg_q3_h, dis_h,
          agg_sh, xs_sh, deg_sh, src_all, dst_all, xbuf, degl, disl, rows,
          ones, sem_a, sem_g, sem_sc):
        cid = lax.axis_index("c")
        tid = lax.axis_index("s")
        r0 = tid * RPT

        @pl.loop(0, RPT // L)
        def _(i):
            degl[pl.ds(i * L, L)] = jnp.zeros((L,), fl)

        @pl.loop(0, EB // L)
        def _(i):
            ones[pl.ds(i * L, L)] = jnp.ones((L,), fl)

        pltpu.sync_copy(src_h.at[pl.ds(tid * nbatch, nbatch)], src_all)
        pltpu.sync_copy(dst_h.at[pl.ds(tid * nbatch, nbatch)], dst_all)

        pltpu.sync_copy(degl, deg_sh.at[pl.ds(r0, RPT)])
        plsc.subcore_barrier()

        for s in range(NSLOT):
            pltpu.async_copy(ones, deg_sh.at[dst_all.at[s]], sem_a, add=True)

        @pl.loop(0, nbatch)
        def _(i):
            @pl.when(i < nbatch - NSLOT)
            def _():
                pltpu.async_copy(ones, deg_sh.at[dst_all.at[i + NSLOT]],
                                 sem_a, add=True)

            pltpu.make_async_copy(ones, deg_sh.at[dst_all.at[0]], sem_a).wait()

        plsc.subcore_barrier()

        pltpu.sync_copy(deg_sh.at[pl.ds(r0, RPT)], degl)

        @pl.loop(0, RPT // L)
        def _(i):
            disl[pl.ds(i * L, L)] = _rsqrt16(degl[pl.ds(i * L, L)])

        @pl.when(cid == 0)
        def _():
            pltpu.sync_copy(disl, dis_h.at[pl.ds(r0, RPT)])

        def do_pass(x_h, agg_out_h):
            HG = NSLOT // 2

            for h in range(4):
                rh = r0 + h * XROWS

                @pl.loop(0, XROWS)
                def _(r):
                    for j in range(DQ // L):
                        xbuf[r, pl.ds(j * L, L)] = jnp.zeros((L,), fl)

                pltpu.sync_copy(xbuf, agg_sh.at[pl.ds(rh, XROWS)])
                pltpu.sync_copy(x_h.at[pl.ds(rh, XROWS)], xbuf)

                @pl.loop(0, XROWS // L)
                def _(i):
                    dv = disl[pl.ds(h * XROWS + i * L, L)]
                    for kk in range(L):
                        s = dv[kk]
                        r = i * L + kk
                        for j in range(DQ // L):
                            xbuf[r, pl.ds(j * L, L)] = (
                                xbuf[r, pl.ds(j * L, L)] * s)

                pltpu.sync_copy(xbuf, xs_sh.at[pl.ds(rh, XROWS)])

            plsc.subcore_barrier()

            def fire_gather(i, s):
                pltpu.async_copy(xs_sh.at[src_all.at[i]], rows[s], sem_g)

            def wait_gather(s):
                pltpu.make_async_copy(xs_sh.at[src_all.at[0]], rows[s],
                                      sem_g).wait()

            def fire_scatter(i, s):
                pltpu.async_copy(rows[s], agg_sh.at[dst_all.at[i]],
                                 sem_sc, add=True)

            def wait_scatter(s):
                pltpu.make_async_copy(rows[s], agg_sh.at[dst_all.at[0]],
                                      sem_sc).wait()

            for s in range(HG):
                fire_gather(s, s)

            @pl.loop(0, nbatch // NSLOT)
            def _(j):
                for s in range(NSLOT):
                    i = j * NSLOT + s
                    wait_gather(s)
                    fire_scatter(i, s)

                    @pl.when(i >= HG)
                    def _():
                        wait_scatter((s + HG) % NSLOT)

                    @pl.when(i < nbatch - HG)
                    def _():
                        fire_gather(i + HG, (s + HG) % NSLOT)

            for s in range(HG):
                wait_scatter(HG + s)

            plsc.subcore_barrier()

            pltpu.sync_copy(agg_sh.at[pl.ds(r0, RPT)],
                            agg_out_h.at[pl.ds(r0, RPT)])
            plsc.subcore_barrier()

        @pl.when(cid == 0)
        def _():
            do_pass(x_q0_h, agg_q0_h)
            do_pass(x_q1_h, agg_q1_h)

        @pl.when(cid == 1)
        def _():
            do_pass(x_q2_h, agg_q2_h)
            do_pass(x_q3_h, agg_q3_h)

    return k(x_q0, x_q1, x_q2, x_q3, src2, dst2)


def _tc_matmul(a0, a1, a2, a3, dis, W, b):
    BM = 256

    def body(a0_ref, a1_ref, a2_ref, a3_ref, dis_ref, w_ref, b_ref, o_ref):
        xc = jnp.concatenate(
            [a0_ref[...], a1_ref[...], a2_ref[...], a3_ref[...]], axis=1
        ) * dis_ref[...]
        o_ref[...] = (
            jnp.dot(xc, w_ref[...], preferred_element_type=jnp.float32)
            + b_ref[...]
        )

    qspec = pl.BlockSpec((BM, DQ), lambda i: (i, 0))
    return pl.pallas_call(
        body,
        grid=(NP // BM,),
        in_specs=[
            qspec, qspec, qspec, qspec,
            pl.BlockSpec((BM, 1), lambda i: (i, 0)),
            pl.BlockSpec((D_IN, D_OUT), lambda i: (0, 0)),
            pl.BlockSpec((1, D_OUT), lambda i: (0, 0)),
        ],
        out_specs=pl.BlockSpec((BM, D_OUT), lambda i: (i, 0)),
        out_shape=jax.ShapeDtypeStruct((NP, D_OUT), jnp.float32),
    )(a0, a1, a2, a3, dis, W, b.reshape(1, D_OUT))


def kernel(x, edge_index, W, b):
    src = edge_index[0].astype(jnp.int32)
    dst = edge_index[1].astype(jnp.int32)
    loop_idx = jnp.arange(N, dtype=jnp.int32)
    e_tot = N + src.shape[0]
    group = TILES * EB * 24
    e_pad = ((e_tot + group - 1) // group) * group
    padv = jnp.full((e_pad - e_tot,), NP - 1, jnp.int32)
    src2 = jnp.concatenate([src, loop_idx, padv]).reshape(e_pad // EB, EB)
    dst2 = jnp.concatenate([dst, loop_idx, padv]).reshape(e_pad // EB, EB)
    x_pad = jnp.zeros((NP, D_IN), jnp.float32).at[:N].set(x)
    a0, a1, a2, a3, dis = _sc_aggregate(
        x_pad[:, 0 * DQ:1 * DQ], x_pad[:, 1 * DQ:2 * DQ],
        x_pad[:, 2 * DQ:3 * DQ], x_pad[:, 3 * DQ:4 * DQ], src2, dst2)
    out = _tc_matmul(a0, a1, a2, a3, dis.reshape(NP, 1), W, b)
    return out[:N]
